# baseline (speedup 1.0000x reference)
import functools

import jax
import jax.numpy as jnp
from jax import lax
from jax.experimental import pallas as pl
from jax.experimental.pallas import tpu as pltpu
from jax.experimental.pallas import tpu_sc as plsc

RING_LO = 4
RING_HI = 103
NSLOT = 112
B, S, E, H = 16, 2048, 256, 64
SCALE = H ** -0.5
LANES = 16
TPL = S // LANES


def _sc_body(x2_hbm, seq_hbm, xg_hbm, valid_hbm,
             seq_v, selp_v, sel_abs_v, validf_v, rows_v, sem):
    nc = 2
    wid = lax.axis_index("s") * nc + lax.axis_index("c")

    @pl.when(wid < B)
    def _():
        b = wid
        lane = lax.iota(jnp.int32, LANES)
        pltpu.sync_copy(seq_hbm.at[b], seq_v)

        for i in range(LANES * NSLOT // LANES):
            selp_v[pl.ds(i * LANES, LANES)] = jnp.full((LANES,), -1, jnp.int32)

        def step(t, carry):
            pos = lane * TPL + t
            v = plsc.load_gather(seq_v, [pos])
            ok = (v >= RING_LO) & (v <= RING_HI)
            slot = jnp.where(ok, v - RING_LO, 0)
            tgt = lane * NSLOT + slot
            plsc.store_scatter(selp_v, [tgt], pos, mask=ok)
            return carry

        lax.fori_loop(0, TPL, step, 0, unroll=4)

        base = b * S
        for jc in range(NSLOT // LANES):
            acc = jnp.full((LANES,), -1, jnp.int32)
            for l in range(LANES):
                acc = jnp.maximum(acc, selp_v[pl.ds(l * NSLOT + jc * LANES, LANES)])
            validf_v[pl.ds(jc * LANES, LANES)] = (acc >= 0).astype(jnp.float32)
            sel_abs_v[pl.ds(jc * LANES, LANES)] = jnp.maximum(acc, 0) + base

        pltpu.async_copy(x2_hbm.at[sel_abs_v], rows_v, sem).wait()
        pltpu.sync_copy(rows_v, xg_hbm.at[b])
        pltpu.sync_copy(validf_v, valid_hbm.at[b])


def _sc_select_gather(x2, seq):
    mesh = plsc.VectorSubcoreMesh(core_axis_name="c", subcore_axis_name="s")
    k = functools.partial(
        pl.kernel,
        mesh=mesh,
        compiler_params=pltpu.CompilerParams(needs_layout_passes=False),
        out_type=[
            jax.ShapeDtypeStruct((B, NSLOT, E), jnp.float32),
            jax.ShapeDtypeStruct((B, NSLOT), jnp.float32),
        ],
        scratch_types=[
            pltpu.VMEM((S,), jnp.int32),
            pltpu.VMEM((LANES * NSLOT,), jnp.int32),
            pltpu.VMEM((NSLOT,), jnp.int32),
            pltpu.VMEM((NSLOT,), jnp.float32),
            pltpu.VMEM((NSLOT, E), jnp.float32),
            pltpu.SemaphoreType.DMA,
        ],
    )(_sc_body)
    return k(x2, seq)


ST = 512
NT = S // ST


def _tc_body(x_ref, xg_ref, valid_ref, w0_ref, b0_ref, w1_ref, b1_ref, out_ref):
    g = jnp.dot(xg_ref[0], w1_ref[...], preferred_element_type=jnp.float32)
    g = g + b1_ref[...]
    out0 = jnp.dot(x_ref[0], w0_ref[...], preferred_element_type=jnp.float32)
    out0 = out0 + b0_ref[...]
    logits = lax.dot_general(
        out0, g, (((1,), (1,)), ((), ())),
        preferred_element_type=jnp.float32)
    logits = logits * valid_ref[0]
    out_ref[0] = SCALE * logits[:, :100]


def kernel(x, sequences, W0, b0, W1, b1):
    x2 = x.reshape(B * S, E)
    xg, valid = _sc_select_gather(x2, sequences)
    valid3 = valid.reshape(B, 1, NSLOT)
    b0r = b0.reshape(1, H)
    b1r = b1.reshape(1, H)
    return pl.pallas_call(
        _tc_body,
        grid=(B, NT),
        in_specs=[
            pl.BlockSpec((1, ST, E), lambda b, t: (b, t, 0)),
            pl.BlockSpec((1, NSLOT, E), lambda b, t: (b, 0, 0)),
            pl.BlockSpec((1, 1, NSLOT), lambda b, t: (b, 0, 0)),
            pl.BlockSpec((E, H), lambda b, t: (0, 0)),
            pl.BlockSpec((1, H), lambda b, t: (0, 0)),
            pl.BlockSpec((E, H), lambda b, t: (0, 0)),
            pl.BlockSpec((1, H), lambda b, t: (0, 0)),
        ],
        out_specs=pl.BlockSpec((1, ST, 100), lambda b, t: (b, t, 0)),
        out_shape=jax.ShapeDtypeStruct((B, S, 100), jnp.float32),
    )(x, xg, valid3, W0, b0r, W1, b1r)


# speedup vs baseline: 1.4699x; 1.4699x over previous
import functools

import jax
import jax.numpy as jnp
from jax import lax
from jax.experimental import pallas as pl
from jax.experimental.pallas import tpu as pltpu
from jax.experimental.pallas import tpu_sc as plsc

RING_LO = 4
RING_HI = 103
NSLOT = 112
B, S, E, H = 16, 2048, 256, 64
SCALE = H ** -0.5
LANES = 16
TPL = S // LANES


def _sc_body(x2_hbm, seq_hbm, xg_hbm, valid_hbm,
             seq_v, selp_v, sel_abs_v, validf_v, rows_v, sem):
    nc = 2
    wid = lax.axis_index("s") * nc + lax.axis_index("c")

    @pl.when(wid < B)
    def _():
        b = wid
        lane = lax.iota(jnp.int32, LANES)
        pltpu.sync_copy(seq_hbm.at[b], seq_v)

        for i in range(LANES * NSLOT // LANES):
            selp_v[pl.ds(i * LANES, LANES)] = jnp.full((LANES,), -1, jnp.int32)

        def step(t, carry):
            v = seq_v[pl.ds(t * LANES, LANES)]
            pos = t * LANES + lane
            ok = (v >= RING_LO) & (v <= RING_HI)
            slot = jnp.where(ok, v - RING_LO, 0)
            tgt = lane * NSLOT + slot
            plsc.store_scatter(selp_v, [tgt], pos, mask=ok)
            return carry

        lax.fori_loop(0, TPL, step, 0, unroll=8)

        base = b * S
        for jc in range(NSLOT // LANES):
            acc = jnp.full((LANES,), -1, jnp.int32)
            for l in range(LANES):
                acc = jnp.maximum(acc, selp_v[pl.ds(l * NSLOT + jc * LANES, LANES)])
            validf_v[pl.ds(jc * LANES, LANES)] = (acc >= 0).astype(jnp.float32)
            sel_abs_v[pl.ds(jc * LANES, LANES)] = jnp.maximum(acc, 0) + base

        pltpu.async_copy(x2_hbm.at[sel_abs_v], rows_v, sem).wait()
        pltpu.sync_copy(rows_v, xg_hbm.at[b])
        pltpu.sync_copy(validf_v, valid_hbm.at[b])


def _sc_select_gather(x2, seq):
    mesh = plsc.VectorSubcoreMesh(core_axis_name="c", subcore_axis_name="s")
    k = functools.partial(
        pl.kernel,
        mesh=mesh,
        compiler_params=pltpu.CompilerParams(needs_layout_passes=False),
        out_type=[
            jax.ShapeDtypeStruct((B, NSLOT, E), jnp.float32),
            jax.ShapeDtypeStruct((B, NSLOT), jnp.float32),
        ],
        scratch_types=[
            pltpu.VMEM((S,), jnp.int32),
            pltpu.VMEM((LANES * NSLOT,), jnp.int32),
            pltpu.VMEM((NSLOT,), jnp.int32),
            pltpu.VMEM((NSLOT,), jnp.float32),
            pltpu.VMEM((NSLOT, E), jnp.float32),
            pltpu.SemaphoreType.DMA,
        ],
    )(_sc_body)
    return k(x2, seq)


ST = 2048
NT = S // ST


def _tc_body(x_ref, xg_ref, valid_ref, w0_ref, b0_ref, w1_ref, b1_ref, out_ref):
    g = jnp.dot(xg_ref[0], w1_ref[...], preferred_element_type=jnp.float32)
    g = g + b1_ref[...]
    out0 = jnp.dot(x_ref[0], w0_ref[...], preferred_element_type=jnp.float32)
    out0 = out0 + b0_ref[...]
    logits = lax.dot_general(
        out0, g, (((1,), (1,)), ((), ())),
        preferred_element_type=jnp.float32)
    logits = logits * valid_ref[0]
    out_ref[0] = SCALE * logits[:, :100]


def kernel(x, sequences, W0, b0, W1, b1):
    x2 = x.reshape(B * S, E)
    xg, valid = _sc_select_gather(x2, sequences)
    valid3 = valid.reshape(B, 1, NSLOT)
    b0r = b0.reshape(1, H)
    b1r = b1.reshape(1, H)
    return pl.pallas_call(
        _tc_body,
        grid=(B, NT),
        in_specs=[
            pl.BlockSpec((1, ST, E), lambda b, t: (b, t, 0)),
            pl.BlockSpec((1, NSLOT, E), lambda b, t: (b, 0, 0)),
            pl.BlockSpec((1, 1, NSLOT), lambda b, t: (b, 0, 0)),
            pl.BlockSpec((E, H), lambda b, t: (0, 0)),
            pl.BlockSpec((1, H), lambda b, t: (0, 0)),
            pl.BlockSpec((E, H), lambda b, t: (0, 0)),
            pl.BlockSpec((1, H), lambda b, t: (0, 0)),
        ],
        out_specs=pl.BlockSpec((1, ST, 100), lambda b, t: (b, t, 0)),
        out_shape=jax.ShapeDtypeStruct((B, S, 100), jnp.float32),
    )(x, xg, valid3, W0, b0r, W1, b1r)


# speedup vs baseline: 2.1985x; 1.4957x over previous
import functools

import jax
import jax.numpy as jnp
from jax import lax
from jax.experimental import pallas as pl
from jax.experimental.pallas import tpu as pltpu
from jax.experimental.pallas import tpu_sc as plsc

RING_LO = 4
RING_HI = 103
NSLOT = 112
B, S, E, H = 16, 2048, 256, 64
SCALE = H ** -0.5
LANES = 16
TPL = S // LANES


def _sc_body(x2_hbm, seq_hbm, xg_hbm, valid_hbm,
             seq_v, selp_v, sel_abs_v, validf_v, rows_v, sem):
    nc = 2
    wid = lax.axis_index("s") * nc + lax.axis_index("c")

    @pl.when(wid < B)
    def _():
        b = wid
        lane = lax.iota(jnp.int32, LANES)
        pltpu.sync_copy(seq_hbm.at[b], seq_v)

        for i in range(NSLOT):
            selp_v[pl.ds(i * LANES, LANES)] = jnp.full((LANES,), -1, jnp.int32)

        def step(t, carry):
            v = seq_v[pl.ds(t * LANES, LANES)]
            pos = t * LANES + lane
            ok = (v >= RING_LO) & (v <= RING_HI)
            slot = jnp.where(ok, v - RING_LO, 0)
            tgt = lane * NSLOT + slot
            plsc.store_scatter(selp_v, [tgt], pos, mask=ok)
            return carry

        lax.fori_loop(0, TPL, step, 0, unroll=8)

        base = b * S
        for jc in range(NSLOT // LANES):
            acc = jnp.full((LANES,), -1, jnp.int32)
            for l in range(LANES):
                acc = jnp.maximum(acc, selp_v[pl.ds(l * NSLOT + jc * LANES, LANES)])
            validf_v[pl.ds(jc * LANES, LANES)] = (acc >= 0).astype(jnp.float32)
            sel_abs_v[pl.ds(jc * LANES, LANES)] = jnp.maximum(acc, 0) + base

        pltpu.async_copy(x2_hbm.at[sel_abs_v], rows_v, sem).wait()
        pltpu.sync_copy(rows_v, xg_hbm.at[b])
        pltpu.sync_copy(validf_v, valid_hbm.at[b])


def _sc_select_gather(x2, seq):
    mesh = plsc.VectorSubcoreMesh(core_axis_name="c", subcore_axis_name="s")
    k = functools.partial(
        pl.kernel,
        mesh=mesh,
        compiler_params=pltpu.CompilerParams(needs_layout_passes=False),
        out_type=[
            jax.ShapeDtypeStruct((B, NSLOT, E), jnp.float32),
            jax.ShapeDtypeStruct((B, NSLOT), jnp.float32),
        ],
        scratch_types=[
            pltpu.VMEM((S,), jnp.int32),
            pltpu.VMEM((LANES * NSLOT,), jnp.int32),
            pltpu.VMEM((NSLOT,), jnp.int32),
            pltpu.VMEM((NSLOT,), jnp.float32),
            pltpu.VMEM((NSLOT, E), jnp.float32),
            pltpu.SemaphoreType.DMA,
        ],
    )(_sc_body)
    return k(x2, seq)


ST = 512
NT = S // ST


def _tc_body(x_ref, xg_ref, valid_ref, w0_ref, b0_ref, w1_ref, b1_ref, out_ref):
    xg2 = xg_ref[...].reshape(B * NSLOT, E)
    g = jnp.dot(xg2, w1_ref[...], preferred_element_type=jnp.float32)
    g = (g + b1_ref[...]).reshape(B, NSLOT, H)
    vt = jnp.transpose(valid_ref[...])

    x2 = x_ref[...].reshape(B * ST, E)
    out0 = jnp.dot(x2, w0_ref[...], preferred_element_type=jnp.float32)
    out0 = out0 + b0_ref[...]

    for b in range(B):
        gb = g[b] * vt[:, b:b + 1]
        o0 = out0[b * ST:(b + 1) * ST, :]
        lt = lax.dot_general(
            gb, o0, (((1,), (1,)), ((), ())),
            preferred_element_type=jnp.float32)
        out_ref[:, b, :] = SCALE * lt[:100, :]


def kernel(x, sequences, W0, b0, W1, b1):
    x2 = x.reshape(B * S, E)
    xg, valid = _sc_select_gather(x2, sequences)
    b0r = b0.reshape(1, H)
    b1r = b1.reshape(1, H)
    outT = pl.pallas_call(
        _tc_body,
        grid=(NT,),
        in_specs=[
            pl.BlockSpec((B, ST, E), lambda t: (0, t, 0)),
            pl.BlockSpec((B, NSLOT, E), lambda t: (0, 0, 0)),
            pl.BlockSpec((B, NSLOT), lambda t: (0, 0)),
            pl.BlockSpec((E, H), lambda t: (0, 0)),
            pl.BlockSpec((1, H), lambda t: (0, 0)),
            pl.BlockSpec((E, H), lambda t: (0, 0)),
            pl.BlockSpec((1, H), lambda t: (0, 0)),
        ],
        out_specs=pl.BlockSpec((100, B, ST), lambda t: (0, 0, t)),
        out_shape=jax.ShapeDtypeStruct((100, B, S), jnp.float32),
    )(x, xg, valid, W0, b0r, W1, b1r)
    return jnp.transpose(outT, (1, 2, 0))


# speedup vs baseline: 2.2036x; 1.0023x over previous
import functools

import jax
import jax.numpy as jnp
from jax import lax
from jax.experimental import pallas as pl
from jax.experimental.pallas import tpu as pltpu
from jax.experimental.pallas import tpu_sc as plsc

RING_LO = 4
RING_HI = 103
NSLOT = 112
B, S, E, H = 16, 2048, 256, 64
SCALE = H ** -0.5
LANES = 16
TPL = S // LANES


def _sc_body(x2_hbm, seq_hbm, xg_hbm, valid_hbm,
             seq_v, selp_v, sel_abs_v, validf_v, rows_v, sem):
    nc = 2
    wid = lax.axis_index("s") * nc + lax.axis_index("c")

    @pl.when(wid < B)
    def _():
        b = wid
        lane = lax.iota(jnp.int32, LANES)
        pltpu.sync_copy(seq_hbm.at[b], seq_v)

        for i in range(NSLOT):
            selp_v[pl.ds(i * LANES, LANES)] = jnp.full((LANES,), -1, jnp.int32)

        def step(t, carry):
            v = seq_v[pl.ds(t * LANES, LANES)]
            pos = t * LANES + lane
            ok = (v >= RING_LO) & (v <= RING_HI)
            slot = jnp.where(ok, v - RING_LO, 0)
            tgt = lane * NSLOT + slot
            plsc.store_scatter(selp_v, [tgt], pos, mask=ok)
            return carry

        lax.fori_loop(0, TPL, step, 0, unroll=8)

        base = b * S
        for jc in range(NSLOT // LANES):
            acc = jnp.full((LANES,), -1, jnp.int32)
            for l in range(LANES):
                acc = jnp.maximum(acc, selp_v[pl.ds(l * NSLOT + jc * LANES, LANES)])
            validf_v[pl.ds(jc * LANES, LANES)] = (acc >= 0).astype(jnp.float32)
            sel_abs_v[pl.ds(jc * LANES, LANES)] = jnp.maximum(acc, 0) + base

        pltpu.async_copy(x2_hbm.at[sel_abs_v], rows_v, sem).wait()
        pltpu.sync_copy(rows_v, xg_hbm.at[b])
        pltpu.sync_copy(validf_v, valid_hbm.at[b])


def _sc_select_gather(x2, seq):
    mesh = plsc.VectorSubcoreMesh(core_axis_name="c", subcore_axis_name="s")
    k = functools.partial(
        pl.kernel,
        mesh=mesh,
        compiler_params=pltpu.CompilerParams(needs_layout_passes=False),
        out_type=[
            jax.ShapeDtypeStruct((B, NSLOT, E), jnp.float32),
            jax.ShapeDtypeStruct((B, NSLOT), jnp.float32),
        ],
        scratch_types=[
            pltpu.VMEM((S,), jnp.int32),
            pltpu.VMEM((LANES * NSLOT,), jnp.int32),
            pltpu.VMEM((NSLOT,), jnp.int32),
            pltpu.VMEM((NSLOT,), jnp.float32),
            pltpu.VMEM((NSLOT, E), jnp.float32),
            pltpu.SemaphoreType.DMA,
        ],
    )(_sc_body)
    return k(x2, seq)


ST = 512
NT = S // ST


def _tc_body(x_ref, xg_ref, valid_ref, w0t_ref, b0_ref, w1t_ref, b1_ref, out_ref):
    xg2 = xg_ref[...].reshape(B * NSLOT, E)
    g = lax.dot_general(xg2, w1t_ref[...], (((1,), (1,)), ((), ())),
                        preferred_element_type=jnp.float32)
    g = (g + b1_ref[...]).reshape(B, NSLOT, H)
    vt = jnp.transpose(valid_ref[...])

    x2 = x_ref[...].reshape(B * ST, E)
    out0 = lax.dot_general(x2, w0t_ref[...], (((1,), (1,)), ((), ())),
                           preferred_element_type=jnp.float32)
    out0 = out0 + b0_ref[...]

    for b in range(B):
        gb = g[b] * vt[:, b:b + 1]
        o0 = out0[b * ST:(b + 1) * ST, :]
        lt = lax.dot_general(
            gb, o0, (((1,), (1,)), ((), ())),
            preferred_element_type=jnp.float32)
        out_ref[:, b, :] = SCALE * lt[:100, :]


def kernel(x, sequences, W0, b0, W1, b1):
    x2 = x.reshape(B * S, E)
    xg, valid = _sc_select_gather(x2, sequences)
    b0r = b0.reshape(1, H)
    b1r = b1.reshape(1, H)
    outT = pl.pallas_call(
        _tc_body,
        grid=(NT,),
        in_specs=[
            pl.BlockSpec((B, ST, E), lambda t: (0, t, 0)),
            pl.BlockSpec((B, NSLOT, E), lambda t: (0, 0, 0)),
            pl.BlockSpec((B, NSLOT), lambda t: (0, 0)),
            pl.BlockSpec((H, E), lambda t: (0, 0)),
            pl.BlockSpec((1, H), lambda t: (0, 0)),
            pl.BlockSpec((H, E), lambda t: (0, 0)),
            pl.BlockSpec((1, H), lambda t: (0, 0)),
        ],
        out_specs=pl.BlockSpec((100, B, ST), lambda t: (0, 0, t)),
        out_shape=jax.ShapeDtypeStruct((100, B, S), jnp.float32),
    )(x, xg, valid, W0.T, b0r, W1.T, b1r)
    return jnp.transpose(outT, (1, 2, 0))


# speedup vs baseline: 2.2394x; 1.0162x over previous
import functools

import jax
import jax.numpy as jnp
from jax import lax
from jax.experimental import pallas as pl
from jax.experimental.pallas import tpu as pltpu
from jax.experimental.pallas import tpu_sc as plsc

RING_LO = 4
RING_HI = 103
NSLOT = 112
B, S, E, H = 16, 2048, 256, 64
SCALE = H ** -0.5
LANES = 16
HALF = S // 2
TPL = HALF // LANES
GROWS = NSLOT // 2


def _sc_body(x2_hbm, seq_hbm, xg_hbm, valid_hbm,
             seq_v, selp_v, sel_v, prt_v, sel_abs_v, validf_v, rows_v,
             shared_sel, sem):
    c = lax.axis_index("c")
    s = lax.axis_index("s")
    b = c * 8 + s // 2
    h = s % 2
    lane = lax.iota(jnp.int32, LANES)

    pltpu.sync_copy(seq_hbm.at[b], seq_v)

    for i in range(NSLOT):
        selp_v[pl.ds(i * LANES, LANES)] = jnp.full((LANES,), -1, jnp.int32)

    off = h * HALF

    def step(t, carry):
        v = seq_v[pl.ds(off + t * LANES, LANES)]
        pos = off + t * LANES + lane
        ok = (v >= RING_LO) & (v <= RING_HI)
        slot = jnp.where(ok, v - RING_LO, 0)
        tgt = lane * NSLOT + slot
        plsc.store_scatter(selp_v, [tgt], pos, mask=ok)
        return carry

    lax.fori_loop(0, TPL, step, 0, unroll=8)

    for jc in range(NSLOT // LANES):
        acc = jnp.full((LANES,), -1, jnp.int32)
        for l in range(LANES):
            acc = jnp.maximum(acc, selp_v[pl.ds(l * NSLOT + jc * LANES, LANES)])
        sel_v[pl.ds(jc * LANES, LANES)] = acc

    pltpu.sync_copy(sel_v, shared_sel.at[s, pl.ds(0, NSLOT)])
    plsc.subcore_barrier()
    pltpu.sync_copy(shared_sel.at[s + 1 - 2 * h, pl.ds(0, NSLOT)], prt_v)

    base = b * S
    for jc in range(NSLOT // LANES):
        acc = jnp.maximum(sel_v[pl.ds(jc * LANES, LANES)],
                          prt_v[pl.ds(jc * LANES, LANES)])
        validf_v[pl.ds(jc * LANES, LANES)] = (acc >= 0).astype(jnp.float32)
        sel_abs_v[pl.ds(jc * LANES, LANES)] = jnp.maximum(acc, 0) + base

    pltpu.async_copy(x2_hbm.at[sel_abs_v.at[pl.ds(h * GROWS, GROWS)]],
                     rows_v, sem).wait()
    pltpu.sync_copy(rows_v, xg_hbm.at[pl.ds(b * NSLOT + h * GROWS, GROWS)])

    @pl.when(h == 0)
    def _():
        pltpu.sync_copy(validf_v, valid_hbm.at[b])


def _sc_select_gather(x2, seq):
    mesh = plsc.VectorSubcoreMesh(core_axis_name="c", subcore_axis_name="s")
    k = functools.partial(
        pl.kernel,
        mesh=mesh,
        compiler_params=pltpu.CompilerParams(needs_layout_passes=False),
        out_type=[
            jax.ShapeDtypeStruct((B * NSLOT, E), jnp.float32),
            jax.ShapeDtypeStruct((B, NSLOT), jnp.float32),
        ],
        scratch_types=[
            pltpu.VMEM((S,), jnp.int32),
            pltpu.VMEM((LANES * NSLOT,), jnp.int32),
            pltpu.VMEM((NSLOT,), jnp.int32),
            pltpu.VMEM((NSLOT,), jnp.int32),
            pltpu.VMEM((NSLOT,), jnp.int32),
            pltpu.VMEM((NSLOT,), jnp.float32),
            pltpu.VMEM((GROWS, E), jnp.float32),
            pltpu.VMEM_SHARED((LANES, 128), jnp.int32),
            pltpu.SemaphoreType.DMA,
        ],
    )(_sc_body)
    xg2, valid = k(x2, seq)
    return xg2.reshape(B, NSLOT, E), valid


ST = 512
NT = S // ST


def _tc_body(x_ref, xg_ref, valid_ref, w0t_ref, b0_ref, w1t_ref, b1_ref, out_ref):
    xg2 = xg_ref[...].reshape(B * NSLOT, E)
    g = lax.dot_general(xg2, w1t_ref[...], (((1,), (1,)), ((), ())),
                        preferred_element_type=jnp.float32)
    g = (g + b1_ref[...]).reshape(B, NSLOT, H)
    vt = jnp.transpose(valid_ref[...])

    x2 = x_ref[...].reshape(B * ST, E)
    out0 = lax.dot_general(x2, w0t_ref[...], (((1,), (1,)), ((), ())),
                           preferred_element_type=jnp.float32)
    out0 = out0 + b0_ref[...]

    for b in range(B):
        gb = g[b] * vt[:, b:b + 1]
        o0 = out0[b * ST:(b + 1) * ST, :]
        lt = lax.dot_general(
            gb, o0, (((1,), (1,)), ((), ())),
            preferred_element_type=jnp.float32)
        out_ref[:, b, :] = SCALE * lt[:100, :]


def kernel(x, sequences, W0, b0, W1, b1):
    x2 = x.reshape(B * S, E)
    xg, valid = _sc_select_gather(x2, sequences)
    b0r = b0.reshape(1, H)
    b1r = b1.reshape(1, H)
    outT = pl.pallas_call(
        _tc_body,
        grid=(NT,),
        in_specs=[
            pl.BlockSpec((B, ST, E), lambda t: (0, t, 0)),
            pl.BlockSpec((B, NSLOT, E), lambda t: (0, 0, 0)),
            pl.BlockSpec((B, NSLOT), lambda t: (0, 0)),
            pl.BlockSpec((H, E), lambda t: (0, 0)),
            pl.BlockSpec((1, H), lambda t: (0, 0)),
            pl.BlockSpec((H, E), lambda t: (0, 0)),
            pl.BlockSpec((1, H), lambda t: (0, 0)),
        ],
        out_specs=pl.BlockSpec((100, B, ST), lambda t: (0, 0, t)),
        out_shape=jax.ShapeDtypeStruct((100, B, S), jnp.float32),
    )(x, xg, valid, W0.T, b0r, W1.T, b1r)
    return jnp.transpose(outT, (1, 2, 0))


# speedup vs baseline: 3.2969x; 1.4722x over previous
import jax
import jax.numpy as jnp
from jax import lax
from jax.experimental import pallas as pl

RING_LO = 4
NSLOT = 112
B, S, E, H = 16, 2048, 256, 64
SCALE = H ** -0.5
BT = 8
NG = B // BT


def _body(x_ref, seq_ref, w0t_ref, b0_ref, w1t_ref, b1_ref, out_ref):
    for b in range(BT):
        xb = x_ref[b]
        seq = seq_ref[b]

        out0 = lax.dot_general(xb, w0t_ref[...], (((1,), (1,)), ((), ())),
                               preferred_element_type=jnp.float32)
        out0 = out0 + b0_ref[...]
        out1 = lax.dot_general(xb, w1t_ref[...], (((1,), (1,)), ((), ())),
                               preferred_element_type=jnp.float32)
        out1 = out1 + b1_ref[...]

        jv = lax.broadcasted_iota(jnp.int32, (NSLOT, S), 0) + RING_LO
        sv = lax.broadcasted_iota(jnp.int32, (NSLOT, S), 1)
        seqb = jnp.broadcast_to(seq, (NSLOT, S))
        cand = jnp.where(seqb == jv, sv, -1)
        sel = jnp.max(cand, axis=1, keepdims=True)
        onehot = ((cand == sel) & (sel >= 0)).astype(jnp.float32)

        slot_rows = jnp.dot(onehot, out1, preferred_element_type=jnp.float32)
        lt = lax.dot_general(slot_rows, out0, (((1,), (1,)), ((), ())),
                             preferred_element_type=jnp.float32)
        out_ref[:, b, :] = SCALE * lt[:100, :]


def kernel(x, sequences, W0, b0, W1, b1):
    seq3 = sequences.reshape(B, 1, S)
    b0r = b0.reshape(1, H)
    b1r = b1.reshape(1, H)
    outT = pl.pallas_call(
        _body,
        grid=(NG,),
        in_specs=[
            pl.BlockSpec((BT, S, E), lambda g: (g, 0, 0)),
            pl.BlockSpec((BT, 1, S), lambda g: (g, 0, 0)),
            pl.BlockSpec((H, E), lambda g: (0, 0)),
            pl.BlockSpec((1, H), lambda g: (0, 0)),
            pl.BlockSpec((H, E), lambda g: (0, 0)),
            pl.BlockSpec((1, H), lambda g: (0, 0)),
        ],
        out_specs=pl.BlockSpec((100, BT, S), lambda g: (0, g, 0)),
        out_shape=jax.ShapeDtypeStruct((100, B, S), jnp.float32),
    )(x, seq3, W0.T, b0r, W1.T, b1r)
    return jnp.transpose(outT, (1, 2, 0))
